# baseline (speedup 1.0000x reference)
import functools

import jax
import jax.numpy as jnp
from jax import lax
from jax.experimental import pallas as pl
from jax.experimental.pallas import tpu as pltpu
from jax.experimental.pallas import tpu_sc as plsc

_TIMESTEPS = 1000.0
_BATCH = 16384
_TABLE = 1001
_TABLE_PAD = 1008
_NC, _NS, _L = 1, 16, 16
_NW = _NC * _NS
_B_PER_W = _BATCH // _NW
_MAGIC = 12582912.0
_UNROLL = 1


_HALF = _B_PER_W // 2


def _body(t_hbm, g_hbm, out_hbm, t_v, g_v, o_v,
          sem_g, sem_t0, sem_t1, sem_o0, sem_o1):
    wid = lax.axis_index("s") * _NC + lax.axis_index("c") if _NC > 1 else lax.axis_index("s")
    base = wid * _B_PER_W
    cp_g = pltpu.async_copy(g_hbm, g_v, sem_g)
    cp_t0 = pltpu.async_copy(
        t_hbm.at[pl.ds(base, _HALF)], t_v.at[pl.ds(0, _HALF)], sem_t0
    )
    cp_t1 = pltpu.async_copy(
        t_hbm.at[pl.ds(base + _HALF, _HALF)], t_v.at[pl.ds(_HALF, _HALF)], sem_t1
    )

    def make_step(chunk_base):
        def step(j, carry):
            base_j = chunk_base + j * (_UNROLL * _L)
            for u in range(_UNROLL):
                off = base_j + u * _L
                tv = t_v[pl.ds(off, _L)]
                r = (tv * _TIMESTEPS + _MAGIC) - _MAGIC
                idx = r.astype(jnp.int32)
                o_v[pl.ds(off, _L)] = plsc.load_gather(g_v, [idx])
            return carry
        return step

    n_steps = _HALF // (_UNROLL * _L)
    cp_g.wait()
    cp_t0.wait()
    lax.fori_loop(0, n_steps, make_step(0), 0)
    cp_o0 = pltpu.async_copy(
        o_v.at[pl.ds(0, _HALF)], out_hbm.at[pl.ds(base, _HALF)], sem_o0
    )
    cp_t1.wait()
    lax.fori_loop(0, n_steps, make_step(_HALF), 0)
    cp_o1 = pltpu.async_copy(
        o_v.at[pl.ds(_HALF, _HALF)], out_hbm.at[pl.ds(base + _HALF, _HALF)], sem_o1
    )
    cp_o0.wait()
    cp_o1.wait()


@jax.jit
def kernel(t, gamma):
    mesh = plsc.VectorSubcoreMesh(
        core_axis_name="c", subcore_axis_name="s", num_cores=_NC
    )
    f = functools.partial(
        pl.kernel,
        mesh=mesh,
        out_type=jax.ShapeDtypeStruct((_BATCH,), jnp.float32),
        scratch_types=[
            pltpu.VMEM((_B_PER_W,), jnp.float32),
            pltpu.VMEM((_TABLE,), jnp.float32),
            pltpu.VMEM((_B_PER_W,), jnp.float32),
            pltpu.SemaphoreType.DMA,
            pltpu.SemaphoreType.DMA,
            pltpu.SemaphoreType.DMA,
            pltpu.SemaphoreType.DMA,
            pltpu.SemaphoreType.DMA,
        ],
        compiler_params=pltpu.CompilerParams(
            needs_layout_passes=False,
            skip_device_barrier=True,
            disable_bounds_checks=True,
            disable_semaphore_checks=True,
        ),
    )(_body)
    return f(t, gamma)


# speedup vs baseline: 1.0004x; 1.0004x over previous
import functools

import jax
import jax.numpy as jnp
from jax import lax
from jax.experimental import pallas as pl
from jax.experimental.pallas import tpu as pltpu
from jax.experimental.pallas import tpu_sc as plsc

_TIMESTEPS = 1000.0
_BATCH = 16384
_TABLE = 1001
_TABLE_PAD = 1008
_NC, _NS, _L = 1, 16, 16
_NW = _NC * _NS
_B_PER_W = _BATCH // _NW
_MAGIC = 12582912.0
_MAGIC_BITS = 0x4B400000
_UNROLL = 2


_HALF = _B_PER_W // 2


def _body(t_hbm, g_hbm, out_hbm, t_v, g_v, o_v,
          sem_g, sem_t0, sem_t1, sem_o0, sem_o1):
    wid = lax.axis_index("s") * _NC + lax.axis_index("c") if _NC > 1 else lax.axis_index("s")
    base = wid * _B_PER_W
    cp_g = pltpu.async_copy(g_hbm, g_v, sem_g)
    cp_t0 = pltpu.async_copy(
        t_hbm.at[pl.ds(base, _HALF)], t_v.at[pl.ds(0, _HALF)], sem_t0
    )
    cp_t1 = pltpu.async_copy(
        t_hbm.at[pl.ds(base + _HALF, _HALF)], t_v.at[pl.ds(_HALF, _HALF)], sem_t1
    )

    def make_step(chunk_base):
        def step(j, carry):
            base_j = chunk_base + j * (_UNROLL * _L)
            for u in range(_UNROLL):
                off = base_j + u * _L
                tv = t_v[pl.ds(off, _L)]
                r = tv * _TIMESTEPS + _MAGIC
                idx = plsc.bitcast(r, jnp.int32) - _MAGIC_BITS
                o_v[pl.ds(off, _L)] = plsc.load_gather(g_v, [idx])
            return carry
        return step

    n_steps = _HALF // (_UNROLL * _L)
    cp_g.wait()
    cp_t0.wait()
    lax.fori_loop(0, n_steps, make_step(0), 0)
    cp_o0 = pltpu.async_copy(
        o_v.at[pl.ds(0, _HALF)], out_hbm.at[pl.ds(base, _HALF)], sem_o0
    )
    cp_t1.wait()
    lax.fori_loop(0, n_steps, make_step(_HALF), 0)
    cp_o1 = pltpu.async_copy(
        o_v.at[pl.ds(_HALF, _HALF)], out_hbm.at[pl.ds(base + _HALF, _HALF)], sem_o1
    )
    cp_o0.wait()
    cp_o1.wait()


@jax.jit
def kernel(t, gamma):
    mesh = plsc.VectorSubcoreMesh(
        core_axis_name="c", subcore_axis_name="s", num_cores=_NC
    )
    f = functools.partial(
        pl.kernel,
        mesh=mesh,
        out_type=jax.ShapeDtypeStruct((_BATCH,), jnp.float32),
        scratch_types=[
            pltpu.VMEM((_B_PER_W,), jnp.float32),
            pltpu.VMEM((_TABLE,), jnp.float32),
            pltpu.VMEM((_B_PER_W,), jnp.float32),
            pltpu.SemaphoreType.DMA,
            pltpu.SemaphoreType.DMA,
            pltpu.SemaphoreType.DMA,
            pltpu.SemaphoreType.DMA,
            pltpu.SemaphoreType.DMA,
        ],
        compiler_params=pltpu.CompilerParams(
            needs_layout_passes=False,
            skip_device_barrier=True,
            disable_bounds_checks=True,
            disable_semaphore_checks=True,
        ),
    )(_body)
    return f(t, gamma)


# speedup vs baseline: 1.0034x; 1.0030x over previous
import functools

import jax
import jax.numpy as jnp
from jax import lax
from jax.experimental import pallas as pl
from jax.experimental.pallas import tpu as pltpu
from jax.experimental.pallas import tpu_sc as plsc

_TIMESTEPS = 1000.0
_BATCH = 16384
_TABLE = 1001
_TABLE_PAD = 1008
_NC, _NS, _L = 1, 16, 16
_NW = _NC * _NS
_B_PER_W = _BATCH // _NW
_MAGIC = 12582912.0
_MAGIC_BITS = 0x4B400000
_UNROLL = 4


_HALF = _B_PER_W // 2


def _body(t_hbm, g_hbm, out_hbm, t_v, g_v, o_v,
          sem_g, sem_t0, sem_t1, sem_o0, sem_o1):
    wid = lax.axis_index("s") * _NC + lax.axis_index("c") if _NC > 1 else lax.axis_index("s")
    base = wid * _B_PER_W
    cp_g = pltpu.async_copy(g_hbm, g_v, sem_g)
    cp_t0 = pltpu.async_copy(
        t_hbm.at[pl.ds(base, _HALF)], t_v.at[pl.ds(0, _HALF)], sem_t0
    )
    cp_t1 = pltpu.async_copy(
        t_hbm.at[pl.ds(base + _HALF, _HALF)], t_v.at[pl.ds(_HALF, _HALF)], sem_t1
    )

    def run_chunk(chunk_base):
        @plsc.parallel_loop(0, _HALF // _L, unroll=_UNROLL)
        def _(j):
            off = chunk_base + j * _L
            tv = t_v[pl.ds(off, _L)]
            r = tv * _TIMESTEPS + _MAGIC
            idx = plsc.bitcast(r, jnp.int32) - _MAGIC_BITS
            o_v[pl.ds(off, _L)] = plsc.load_gather(g_v, [idx])

    cp_g.wait()
    cp_t0.wait()
    run_chunk(0)
    cp_o0 = pltpu.async_copy(
        o_v.at[pl.ds(0, _HALF)], out_hbm.at[pl.ds(base, _HALF)], sem_o0
    )
    cp_t1.wait()
    run_chunk(_HALF)
    cp_o1 = pltpu.async_copy(
        o_v.at[pl.ds(_HALF, _HALF)], out_hbm.at[pl.ds(base + _HALF, _HALF)], sem_o1
    )
    cp_o0.wait()
    cp_o1.wait()


@jax.jit
def kernel(t, gamma):
    mesh = plsc.VectorSubcoreMesh(
        core_axis_name="c", subcore_axis_name="s", num_cores=_NC
    )
    f = functools.partial(
        pl.kernel,
        mesh=mesh,
        out_type=jax.ShapeDtypeStruct((_BATCH,), jnp.float32),
        scratch_types=[
            pltpu.VMEM((_B_PER_W,), jnp.float32),
            pltpu.VMEM((_TABLE,), jnp.float32),
            pltpu.VMEM((_B_PER_W,), jnp.float32),
            pltpu.SemaphoreType.DMA,
            pltpu.SemaphoreType.DMA,
            pltpu.SemaphoreType.DMA,
            pltpu.SemaphoreType.DMA,
            pltpu.SemaphoreType.DMA,
        ],
        compiler_params=pltpu.CompilerParams(
            needs_layout_passes=False,
            skip_device_barrier=True,
            disable_bounds_checks=True,
            disable_semaphore_checks=True,
        ),
    )(_body)
    return f(t, gamma)


# speedup vs baseline: 1.0136x; 1.0102x over previous
import functools

import jax
import jax.numpy as jnp
from jax import lax
from jax.experimental import pallas as pl
from jax.experimental.pallas import tpu as pltpu
from jax.experimental.pallas import tpu_sc as plsc

_TIMESTEPS = 1000.0
_BATCH = 16384
_TABLE = 1001
_TABLE_PAD = 1008
_NC, _NS, _L = 1, 16, 16
_NW = _NC * _NS
_B_PER_W = _BATCH // _NW
_MAGIC = 12582912.0
_MAGIC_BITS = 0x4B400000
_UNROLL = 4


def _body(t_hbm, g_hbm, out_hbm, x_v, g_v, sem_g, sem_x):
    base = lax.axis_index("s") * _B_PER_W
    cp_g = pltpu.async_copy(g_hbm, g_v, sem_g)
    cp_t = pltpu.async_copy(t_hbm.at[pl.ds(base, _B_PER_W)], x_v, sem_x)
    cp_g.wait()
    cp_t.wait()

    @plsc.parallel_loop(0, _B_PER_W // _L, unroll=_UNROLL)
    def _(j):
        off = j * _L
        tv = x_v[pl.ds(off, _L)]
        r = tv * _TIMESTEPS + _MAGIC
        idx = plsc.bitcast(r, jnp.int32) - _MAGIC_BITS
        x_v[pl.ds(off, _L)] = plsc.load_gather(g_v, [idx])

    pltpu.sync_copy(x_v, out_hbm.at[pl.ds(base, _B_PER_W)])


@jax.jit
def kernel(t, gamma):
    mesh = plsc.VectorSubcoreMesh(
        core_axis_name="c", subcore_axis_name="s", num_cores=_NC
    )
    f = functools.partial(
        pl.kernel,
        mesh=mesh,
        out_type=jax.ShapeDtypeStruct((_BATCH,), jnp.float32),
        scratch_types=[
            pltpu.VMEM((_B_PER_W,), jnp.float32),
            pltpu.VMEM((_TABLE,), jnp.float32),
            pltpu.SemaphoreType.DMA,
            pltpu.SemaphoreType.DMA,
        ],
        compiler_params=pltpu.CompilerParams(
            needs_layout_passes=False,
            skip_device_barrier=True,
            disable_bounds_checks=True,
            disable_semaphore_checks=True,
        ),
    )(_body)
    return f(t, gamma)
